# baseline (speedup 1.0000x reference)
import functools

import jax
import jax.numpy as jnp
from jax import lax
from jax.experimental import pallas as pl
from jax.experimental.pallas import tpu as pltpu
from jax.experimental.pallas import tpu_sc as plsc


@functools.partial(jax.jit, static_argnames=("b", "l", "d"))
def _sc_gather(table, idx, b, l, d):
    info = plsc.get_sparse_core_info()
    nc, ns, nl = info.num_cores, info.num_subcores, info.num_lanes
    nw = nc * ns
    assert b % (nw * 128) == 0 and d % 8 == 0
    bw = b // nw
    nb0 = bw // 128
    nd0 = d // 8
    per_w = bw * l
    kq = d // nl

    mesh = plsc.VectorSubcoreMesh(core_axis_name="c", subcore_axis_name="s")

    @functools.partial(
        pl.kernel,
        mesh=mesh,
        out_type=jax.ShapeDtypeStruct(
            (l, nd0, b // 128, 8, 128), jnp.float32),
        compiler_params=pltpu.CompilerParams(
            use_tc_tiling_on_sc=False, needs_layout_passes=False),
        scratch_types=[
            pltpu.VMEM((per_w,), jnp.int32),
            pltpu.VMEM((bw,), jnp.int32),
            pltpu.VMEM((bw,), jnp.int32),
            pltpu.VMEM((bw, d), jnp.float32),
            pltpu.VMEM((bw, d), jnp.float32),
            pltpu.VMEM((nd0, nb0, 8, 128), jnp.float32),
            pltpu.SemaphoreType.DMA,
            pltpu.SemaphoreType.DMA,
            pltpu.SemaphoreType.DMA,
        ],
    )
    def k(table_hbm, idx_hbm, z_hbm, idx_all, il0, il1, r0, r1, stg,
          g0, g1, wsem):
        wid = lax.axis_index("s") * nc + lax.axis_index("c")
        idx_v = (il0, il1)
        rows_v = (r0, r1)
        gsem = (g0, g1)

        pltpu.sync_copy(idx_hbm.at[pl.ds(wid * per_w, per_w)], idx_all)

        lane = jnp.arange(nl, dtype=jnp.int32)
        lane_l = lane * l
        d0b = lane // 8
        dsb = lane % 8

        def build_idx(li, bb):
            for q in range(bw // nl):
                addr = lane_l + (q * nl * l + li)
                v = plsc.load_gather(idx_all, [addr])
                idx_v[bb][pl.ds(q * nl, nl)] = v

        def start_gather(bb):
            pltpu.async_copy(table_hbm.at[idx_v[bb]], rows_v[bb], gsem[bb])

        def wait_gather(bb):
            pltpu.make_async_copy(
                table_hbm.at[idx_v[bb]], rows_v[bb], gsem[bb]).wait()

        def out_dma(li, start):
            for d0 in range(nd0):
                cp = (pltpu.async_copy if start else pltpu.make_async_copy)
                r = cp(stg.at[d0],
                       z_hbm.at[li, d0, pl.ds(nb0 * wid, nb0)], wsem)
                if not start:
                    r.wait()

        d0qs = [d0b + 2 * q for q in range(kq)]

        def transpose(bb):
            def body(bp):
                b0f = jnp.full((nl,), bp // 128, dtype=jnp.int32)
                bsf = jnp.full((nl,), bp % 128, dtype=jnp.int32)
                for q in range(kq):
                    vec = rows_v[bb][bp, pl.ds(q * nl, nl)]
                    plsc.store_scatter(stg, [d0qs[q], b0f, dsb, bsf], vec)
            pl.loop(0, bw, unroll=8)(body)

        build_idx(0, 0)
        start_gather(0)

        def phase(li, bb):
            nbb = 1 - bb
            wait_gather(bb)

            @pl.when(li < l - 1)
            def _():
                build_idx(li + 1, nbb)
                start_gather(nbb)

            @pl.when(li > 0)
            def _():
                out_dma(li - 1, start=False)

            transpose(bb)
            out_dma(li, start=True)

        def step(i, carry):
            phase(2 * i, 0)
            phase(2 * i + 1, 1)
            return carry

        lax.fori_loop(0, l // 2, step, 0)
        out_dma(l - 1, start=False)

    return k(table, idx)


def kernel(text_vec, w2v_table):
    b, l = text_vec.shape
    v, d = w2v_table.shape
    idx = text_vec.reshape(b * l).astype(jnp.int32)
    z = _sc_gather(w2v_table, idx, b, l, d)
    return jnp.transpose(z, (2, 4, 0, 1, 3)).reshape(b, l, d)


# speedup vs baseline: 1.6026x; 1.6026x over previous
import functools

import jax
import jax.numpy as jnp
from jax import lax
from jax.experimental import pallas as pl
from jax.experimental.pallas import tpu as pltpu
from jax.experimental.pallas import tpu_sc as plsc


@functools.partial(jax.jit, static_argnames=("b", "l", "d"))
def _sc_gather(table, idx, b, l, d):
    info = plsc.get_sparse_core_info()
    nc, ns, nl = info.num_cores, info.num_subcores, info.num_lanes
    nw = nc * ns
    assert b % (nw * 128) == 0 and d % 8 == 0
    bw = b // nw
    nb0 = bw // 128
    nd0 = d // 8
    per_w = bw * l
    kq = d // nl

    mesh = plsc.VectorSubcoreMesh(core_axis_name="c", subcore_axis_name="s")

    @functools.partial(
        pl.kernel,
        mesh=mesh,
        out_type=jax.ShapeDtypeStruct(
            (l, nd0, b // 128, 8, 128), jnp.float32),
        compiler_params=pltpu.CompilerParams(
            use_tc_tiling_on_sc=False, needs_layout_passes=False),
        scratch_types=[
            pltpu.VMEM((per_w,), jnp.int32),
            pltpu.VMEM((bw,), jnp.int32),
            pltpu.VMEM((bw,), jnp.int32),
            pltpu.VMEM((bw, d), jnp.float32),
            pltpu.VMEM((bw, d), jnp.float32),
            pltpu.VMEM((nd0, nb0, 8, 129), jnp.float32),
            pltpu.SemaphoreType.DMA,
            pltpu.SemaphoreType.DMA,
            pltpu.SemaphoreType.DMA,
        ],
    )
    def k(table_hbm, idx_hbm, z_hbm, idx_all, il0, il1, r0, r1, stg,
          g0, g1, wsem):
        wid = lax.axis_index("s") * nc + lax.axis_index("c")
        idx_v = (il0, il1)
        rows_v = (r0, r1)
        gsem = (g0, g1)

        pltpu.sync_copy(idx_hbm.at[pl.ds(wid * per_w, per_w)], idx_all)

        lane = jnp.arange(nl, dtype=jnp.int32)
        lane_l = lane * l
        d0b = lane // 8
        dsb = lane % 8

        def build_idx(li, bb):
            for q in range(bw // nl):
                addr = lane_l + (q * nl * l + li)
                v = plsc.load_gather(idx_all, [addr])
                idx_v[bb][pl.ds(q * nl, nl)] = v

        def start_gather(bb):
            pltpu.async_copy(table_hbm.at[idx_v[bb]], rows_v[bb], gsem[bb])

        def wait_gather(bb):
            pltpu.make_async_copy(
                table_hbm.at[idx_v[bb]], rows_v[bb], gsem[bb]).wait()

        def out_dma(li, start):
            for d0 in range(nd0):
                cp = (pltpu.async_copy if start else pltpu.make_async_copy)
                r = cp(stg.at[d0, :, pl.ds(0, 8), pl.ds(0, 128)],
                       z_hbm.at[li, d0, pl.ds(nb0 * wid, nb0)], wsem)
                if not start:
                    r.wait()

        d0qs = [d0b + 2 * q for q in range(kq)]

        def transpose(bb):
            def body(bp):
                b0f = jnp.full((nl,), bp // 128, dtype=jnp.int32)
                bsf = jnp.full((nl,), bp % 128, dtype=jnp.int32)
                for q in range(kq):
                    vec = rows_v[bb][bp, pl.ds(q * nl, nl)]
                    plsc.store_scatter(stg, [d0qs[q], b0f, dsb, bsf], vec)
            pl.loop(0, bw, unroll=8)(body)

        build_idx(0, 0)
        start_gather(0)

        def phase(li, bb):
            nbb = 1 - bb
            wait_gather(bb)

            @pl.when(li < l - 1)
            def _():
                build_idx(li + 1, nbb)
                start_gather(nbb)

            @pl.when(li > 0)
            def _():
                out_dma(li - 1, start=False)

            transpose(bb)
            out_dma(li, start=True)

        def step(i, carry):
            phase(2 * i, 0)
            phase(2 * i + 1, 1)
            return carry

        lax.fori_loop(0, l // 2, step, 0)
        out_dma(l - 1, start=False)

    return k(table, idx)


def kernel(text_vec, w2v_table):
    b, l = text_vec.shape
    v, d = w2v_table.shape
    idx = text_vec.reshape(b * l).astype(jnp.int32)
    z = _sc_gather(w2v_table, idx, b, l, d)
    return jnp.transpose(z, (2, 4, 0, 1, 3)).reshape(b, l, d)


# speedup vs baseline: 1.6152x; 1.0079x over previous
import functools

import jax
import jax.numpy as jnp
from jax import lax
from jax.experimental import pallas as pl
from jax.experimental.pallas import tpu as pltpu
from jax.experimental.pallas import tpu_sc as plsc


@functools.partial(jax.jit, static_argnames=("b", "l", "d"))
def _sc_gather(table, idx, b, l, d):
    info = plsc.get_sparse_core_info()
    nc, ns, nl = info.num_cores, info.num_subcores, info.num_lanes
    nw = nc * ns
    assert b % (nw * 128) == 0 and d % 8 == 0
    bw = b // nw
    nb0 = bw // 128
    nd0 = d // 8
    per_w = bw * l
    kq = d // nl

    mesh = plsc.VectorSubcoreMesh(core_axis_name="c", subcore_axis_name="s")

    @functools.partial(
        pl.kernel,
        mesh=mesh,
        out_type=jax.ShapeDtypeStruct(
            (l, nd0, b // 128, 8, 128), jnp.float32),
        compiler_params=pltpu.CompilerParams(
            use_tc_tiling_on_sc=False, needs_layout_passes=False),
        scratch_types=[
            pltpu.VMEM((per_w,), jnp.int32),
            pltpu.VMEM((bw,), jnp.int32),
            pltpu.VMEM((bw,), jnp.int32),
            pltpu.VMEM((bw, d), jnp.float32),
            pltpu.VMEM((bw, d), jnp.float32),
            pltpu.VMEM((nb0, nd0, 8, 129), jnp.float32),
            pltpu.SemaphoreType.DMA,
            pltpu.SemaphoreType.DMA,
            pltpu.SemaphoreType.DMA,
        ],
    )
    def k(table_hbm, idx_hbm, z_hbm, idx_all, il0, il1, r0, r1, stg,
          g0, g1, wsem):
        wid = lax.axis_index("s") * nc + lax.axis_index("c")
        idx_v = (il0, il1)
        rows_v = (r0, r1)
        gsem = (g0, g1)

        pltpu.sync_copy(idx_hbm.at[pl.ds(wid * per_w, per_w)], idx_all)

        lane = jnp.arange(nl, dtype=jnp.int32)
        lane_l = lane * l
        d0b = lane // 8
        dsb = lane % 8

        def build_idx(li, bb):
            for q in range(bw // nl):
                addr = lane_l + (q * nl * l + li)
                v = plsc.load_gather(idx_all, [addr])
                idx_v[bb][pl.ds(q * nl, nl)] = v

        def start_gather(bb):
            pltpu.async_copy(table_hbm.at[idx_v[bb]], rows_v[bb], gsem[bb])

        def wait_gather(bb):
            pltpu.make_async_copy(
                table_hbm.at[idx_v[bb]], rows_v[bb], gsem[bb]).wait()

        def out_dma(li, start):
            for d0 in range(nd0):
                cp = (pltpu.async_copy if start else pltpu.make_async_copy)
                r = cp(stg.at[:, d0, pl.ds(0, 8), pl.ds(0, 128)],
                       z_hbm.at[li, d0, pl.ds(nb0 * wid, nb0)], wsem)
                if not start:
                    r.wait()

        d0qs = [d0b + 2 * q for q in range(kq)]

        def transpose(bb):
            def body(bp):
                b0f = jnp.full((nl,), bp // 128, dtype=jnp.int32)
                bsf = jnp.full((nl,), bp % 128, dtype=jnp.int32)
                for q in range(kq):
                    vec = rows_v[bb][bp, pl.ds(q * nl, nl)]
                    plsc.store_scatter(stg, [b0f, d0qs[q], dsb, bsf], vec)
            pl.loop(0, bw, unroll=8)(body)

        build_idx(0, 0)
        start_gather(0)

        def phase(li, bb):
            nbb = 1 - bb
            wait_gather(bb)

            @pl.when(li < l - 1)
            def _():
                build_idx(li + 1, nbb)
                start_gather(nbb)

            @pl.when(li > 0)
            def _():
                out_dma(li - 1, start=False)

            transpose(bb)
            out_dma(li, start=True)

        def step(i, carry):
            phase(2 * i, 0)
            phase(2 * i + 1, 1)
            return carry

        lax.fori_loop(0, l // 2, step, 0)
        out_dma(l - 1, start=False)

    return k(table, idx)


def kernel(text_vec, w2v_table):
    b, l = text_vec.shape
    v, d = w2v_table.shape
    idx = text_vec.reshape(b * l).astype(jnp.int32)
    z = _sc_gather(w2v_table, idx, b, l, d)
    return jnp.transpose(z, (2, 4, 0, 1, 3)).reshape(b, l, d)
